# Initial kernel scaffold; baseline (speedup 1.0000x reference)
import jax
import jax.numpy as jnp
from jax.experimental import pallas as pl


def kernel(x, edge_index, batch, W1, b1, g1, be1, W2, b2, g2, be2):
    raise NotImplementedError("write your pallas kernel here")


# speedup vs baseline: 3.4728x; 3.4728x over previous
import functools

import jax
import jax.numpy as jnp
from jax import lax
from jax.experimental import pallas as pl
from jax.experimental.pallas import tpu as pltpu
from jax.experimental.pallas import tpu_sc as plsc

_EPS = 1e-5


def _make_sc_segsum(n_pad, half, nchunk, chunk, nsub):
    rows_per_tile = n_pad // nsub
    ncopy = rows_per_tile // chunk

    mesh = plsc.VectorSubcoreMesh(core_axis_name="c", subcore_axis_name="s")

    @functools.partial(
        pl.kernel,
        mesh=mesh,
        out_type=jax.ShapeDtypeStruct((2, n_pad, half), jnp.float32),
        scratch_types=[
            pltpu.VMEM((nchunk, chunk), jnp.int32),
            pltpu.VMEM((nchunk, chunk), jnp.int32),
            pltpu.VMEM((chunk, half), jnp.float32),
            pltpu.VMEM_SHARED((n_pad, half), jnp.float32),
            pltpu.SemaphoreType.DMA,
        ],
    )
    def segsum(hcat_hbm, src_hbm, dst_hbm, out_hbm, src_v, dst_v, rows_v,
               acc_sh, sem):
        c = lax.axis_index("c")
        s = lax.axis_index("s")
        base = s * rows_per_tile

        def _zrow(i, _):
            r = i // (half // 16)
            k = i % (half // 16)
            rows_v[r, pl.ds(k * 16, 16)] = jnp.zeros((16,), jnp.float32)
            return 0

        lax.fori_loop(0, chunk * (half // 16), _zrow, 0)
        for r in range(ncopy):
            pltpu.sync_copy(rows_v, acc_sh.at[pl.ds(base + r * chunk, chunk)])
        plsc.subcore_barrier()

        pltpu.sync_copy(src_hbm.at[c, s], src_v)
        pltpu.sync_copy(dst_hbm.at[s], dst_v)

        def _edge_chunk(j, _):
            pltpu.async_copy(hcat_hbm.at[src_v.at[j]], rows_v, sem).wait()
            pltpu.sync_copy(rows_v, acc_sh.at[dst_v.at[j]], add=True)
            return 0

        lax.fori_loop(0, nchunk, _edge_chunk, 0)
        plsc.subcore_barrier()

        pltpu.sync_copy(acc_sh.at[pl.ds(base, rows_per_tile)],
                        out_hbm.at[c, pl.ds(base, rows_per_tile)])

    return segsum


def _dot3(z, w):
    return jnp.dot(z.astype(jnp.bfloat16), w.astype(jnp.bfloat16),
                   preferred_element_type=jnp.float32)


def _stats_update(st_r, y, i, tile, d):
    m_i = jnp.mean(y, axis=0, keepdims=True)
    yc = y - m_i
    m2_i = jnp.sum(yc * yc, axis=0, keepdims=True)
    pad = jnp.zeros((6, d), jnp.float32)

    @pl.when(i == 0)
    def _():
        st_r[...] = jnp.concatenate([m_i, m2_i, pad], axis=0)

    @pl.when(i > 0)
    def _():
        st = st_r[...]
        n_a = (i * tile).astype(jnp.float32)
        n_tot = n_a + tile
        delta = m_i - st[0:1]
        mean = st[0:1] + delta * (tile / n_tot)
        m2 = st[1:2] + m2_i + delta * delta * (n_a * tile / n_tot)
        st_r[...] = jnp.concatenate([mean, m2, pad], axis=0)

def _p1(h3, m3, w, b, rows, tile):
    grid = rows // tile
    half = h3.shape[2]
    d = w.shape[1]

    def body(ha_r, hb_r, ma_r, mb_r, w_r, b_r, y_r, st_r):
        i = pl.program_id(0)
        z = jnp.concatenate([ha_r[...] + ma_r[...], hb_r[...] + mb_r[...]],
                            axis=1)
        y = _dot3(z, w_r[...]) + b_r[...]
        y_r[...] = y
        _stats_update(st_r, y, i, tile, d)

    return pl.pallas_call(
        body,
        grid=(grid,),
        in_specs=[
            pl.BlockSpec((None, tile, half), lambda i: (0, i, 0)),
            pl.BlockSpec((None, tile, half), lambda i: (1, i, 0)),
            pl.BlockSpec((None, tile, half), lambda i: (0, i, 0)),
            pl.BlockSpec((None, tile, half), lambda i: (1, i, 0)),
            pl.BlockSpec((2 * half, d), lambda i: (0, 0)),
            pl.BlockSpec((1, d), lambda i: (0, 0)),
        ],
        out_specs=[
            pl.BlockSpec((tile, d), lambda i: (i, 0)),
            pl.BlockSpec((8, d), lambda i: (0, 0)),
        ],
        out_shape=[
            jax.ShapeDtypeStruct((rows, d), jnp.float32),
            jax.ShapeDtypeStruct((8, d), jnp.float32),
        ],
    )(h3, h3, m3, m3, w, b)


def _p2(y1, st1, g, be, w, b, rows, tile):
    grid = rows // tile
    d = y1.shape[1]
    inv_n = 1.0 / rows

    def body(y1_r, st_r, g_r, be_r, w_r, b_r, y2_r, st2_r):
        i = pl.program_id(0)
        st = st_r[...]
        mu = st[0:1]
        var = st[1:2] * inv_n
        s = jnp.sqrt(var + _EPS)
        a = jnp.maximum((y1_r[...] - mu) / s * g_r[...] + be_r[...], 0.0)
        y2 = _dot3(a, w_r[...]) + b_r[...]
        y2_r[...] = y2
        _stats_update(st2_r, y2, i, tile, d)

    return pl.pallas_call(
        body,
        grid=(grid,),
        in_specs=[
            pl.BlockSpec((tile, d), lambda i: (i, 0)),
            pl.BlockSpec((8, d), lambda i: (0, 0)),
            pl.BlockSpec((1, d), lambda i: (0, 0)),
            pl.BlockSpec((1, d), lambda i: (0, 0)),
            pl.BlockSpec((d, d), lambda i: (0, 0)),
            pl.BlockSpec((1, d), lambda i: (0, 0)),
        ],
        out_specs=[
            pl.BlockSpec((tile, d), lambda i: (i, 0)),
            pl.BlockSpec((8, d), lambda i: (0, 0)),
        ],
        out_shape=[
            jax.ShapeDtypeStruct((rows, d), jnp.float32),
            jax.ShapeDtypeStruct((8, d), jnp.float32),
        ],
    )(y1, st1, g, be, w, b)


def _p3_split(y2, st2, g, be, rows, tile):
    grid = rows // tile
    d = y2.shape[1]
    half = d // 2
    inv_n = 1.0 / rows

    def body(y2_r, st_r, g_r, be_r, h3_r):
        st = st_r[...]
        mu = st[0:1]
        var = st[1:2] * inv_n
        s = jnp.sqrt(var + _EPS)
        h = jnp.maximum((y2_r[...] - mu) / s * g_r[...] + be_r[...], 0.0)
        h3_r[0] = h[:, :half]
        h3_r[1] = h[:, half:]

    return pl.pallas_call(
        body,
        grid=(grid,),
        in_specs=[
            pl.BlockSpec((tile, d), lambda i: (i, 0)),
            pl.BlockSpec((8, d), lambda i: (0, 0)),
            pl.BlockSpec((1, d), lambda i: (0, 0)),
            pl.BlockSpec((1, d), lambda i: (0, 0)),
        ],
        out_specs=pl.BlockSpec((2, tile, half), lambda i: (0, i, 0)),
        out_shape=jax.ShapeDtypeStruct((2, rows, half), jnp.float32),
    )(y2, st2, g, be)


def _p3_final(y2, st2, g, be, rows, tile):
    grid = rows // tile
    d = y2.shape[1]
    inv_n = 1.0 / rows

    def body(y2_r, st_r, g_r, be_r, out_r):
        st = st_r[...]
        mu = st[0:1]
        var = st[1:2] * inv_n
        s = jnp.sqrt(var + _EPS)
        out_r[...] = (y2_r[...] - mu) / s * g_r[...] + be_r[...]

    return pl.pallas_call(
        body,
        grid=(grid,),
        in_specs=[
            pl.BlockSpec((tile, d), lambda i: (i, 0)),
            pl.BlockSpec((8, d), lambda i: (0, 0)),
            pl.BlockSpec((1, d), lambda i: (0, 0)),
            pl.BlockSpec((1, d), lambda i: (0, 0)),
        ],
        out_specs=pl.BlockSpec((tile, d), lambda i: (i, 0)),
        out_shape=jax.ShapeDtypeStruct((rows, d), jnp.float32),
    )(y2, st2, g, be)


def kernel(x, edge_index, batch, W1, b1, g1, be1, W2, b2, g2, be2):
    del batch
    n, d = x.shape
    e = edge_index.shape[1]
    nlayer = W1.shape[0]
    half = d // 2

    nsub = 16
    chunk = 128
    grain = nsub * chunk
    nchunk = -(-e // grain)
    e_pad = nchunk * grain
    n_pad = -(-n // (nsub * chunk)) * (nsub * chunk)

    src = edge_index[0].astype(jnp.int32)
    dst = edge_index[1].astype(jnp.int32)
    src = jnp.concatenate([src, jnp.zeros((e_pad - e,), jnp.int32)])
    dst = jnp.concatenate([dst, jnp.full((e_pad - e,), n, jnp.int32)])
    src3 = src.reshape(nsub, nchunk, chunk)
    src4 = jnp.stack([src3, src3 + n])
    dst3 = dst.reshape(nsub, nchunk, chunk)

    segsum = _make_sc_segsum(n_pad, half, nchunk, chunk, nsub)

    tile = 2000
    h3 = jnp.stack([x[:, :half], x[:, half:]]).astype(jnp.float32)
    for l in range(nlayer):
        m3 = segsum(h3.reshape(2 * n, half), src4, dst3)
        y1, s1 = _p1(h3, m3, W1[l], b1[l][None], n, tile)
        y2, s2 = _p2(y1, s1, g1[l][None], be1[l][None], W2[l], b2[l][None], n, tile)
        if l < nlayer - 1:
            h3 = _p3_split(y2, s2, g2[l][None], be2[l][None], n, tile)
        else:
            out = _p3_final(y2, s2, g2[l][None], be2[l][None], n, tile)
    return out
